# baseline (speedup 1.0000x reference)
import functools

import jax
import jax.numpy as jnp
from jax import lax
from jax.experimental import pallas as pl
from jax.experimental.pallas import tpu as pltpu
from jax.experimental.pallas import tpu_sc as plsc

_NUM_CORES = 2
_NUM_SUBCORES = 16
_NUM_WORKERS = _NUM_CORES * _NUM_SUBCORES
_LANES = 16


def _dequant_kernel(idx_hbm, w_hbm, s_hbm, out_hbm,
                    idx_v, w_v0, w_v1, s_v0, s_v1, out_v0, out_v1,
                    sem_g0, sem_g1, sem_o0, sem_o1):
  hist, batch = idx_hbm.shape
  d8 = out_hbm.shape[1]
  bm = out_hbm.shape[4]
  pairs = hist // 2

  wid = lax.axis_index("s") * _NUM_CORES + lax.axis_index("c")
  b0 = wid * bm

  iota = lax.iota(jnp.int32, _LANES)
  row_hi = iota >> 1
  row_lo = [(iota & 1) * 4 + j for j in range(4)]

  pltpu.sync_copy(idx_hbm.at[:, pl.ds(b0, bm)], idx_v)

  def fire_gather(l, w_v, s_v, sem):
    cw = pltpu.async_copy(w_hbm.at[idx_v.at[l]], w_v, sem)
    cs = pltpu.async_copy(s_hbm.at[idx_v.at[l]], s_v.at[pl.ds(0, bm)], sem)
    return cw, cs

  def compute(l, w_v, s_v, out_v, sem_o):
    def row_body(r, _):
      packed = w_v[r, 0]
      words = plsc.bitcast(packed, jnp.int32)
      s_vec = s_v[pl.ds(r, _LANES)]
      s = jnp.broadcast_to(s_vec[0], (_LANES,))
      r_splat = jnp.full((_LANES,), r, jnp.int32)
      for j in range(4):
        v = (words << (24 - 8 * j)) >> 24 if j < 3 else words >> 24
        plsc.store_scatter(out_v, [row_hi, row_lo[j], r_splat],
                           v.astype(jnp.float32) * s)
      return 0

    lax.fori_loop(0, bm, row_body, 0)
    for k in range(d8):
      pltpu.async_copy(out_v.at[k], out_hbm.at[l, k, wid], sem_o)

  def drain_out(out_v, sem_o):
    for k in range(d8):
      pltpu.make_async_copy(out_v.at[k], out_hbm.at[0, k, wid], sem_o).wait()

  g0 = fire_gather(0, w_v0, s_v0, sem_g0)

  def pair_body(t, _):
    l0 = 2 * t
    l1 = l0 + 1
    g1 = fire_gather(l1, w_v1, s_v1, sem_g1)
    pltpu.make_async_copy(w_hbm.at[idx_v.at[0]], w_v0, sem_g0).wait()
    pltpu.make_async_copy(
        s_hbm.at[idx_v.at[0]], s_v0.at[pl.ds(0, bm)], sem_g0).wait()

    @pl.when(t > 0)
    def _():
      drain_out(out_v0, sem_o0)

    compute(l0, w_v0, s_v0, out_v0, sem_o0)

    @pl.when(t < pairs - 1)
    def _():
      fire_gather(l0 + 2, w_v0, s_v0, sem_g0)

    pltpu.make_async_copy(w_hbm.at[idx_v.at[0]], w_v1, sem_g1).wait()
    pltpu.make_async_copy(
        s_hbm.at[idx_v.at[0]], s_v1.at[pl.ds(0, bm)], sem_g1).wait()

    @pl.when(t > 0)
    def _():
      drain_out(out_v1, sem_o1)

    compute(l1, w_v1, s_v1, out_v1, sem_o1)
    return 0

  lax.fori_loop(0, pairs, pair_body, 0)
  drain_out(out_v0, sem_o0)
  drain_out(out_v1, sem_o1)


def kernel(input, weight_int8, scale):
  batch, hist = input.shape
  vocab, dim = weight_int8.shape
  bm = batch // _NUM_WORKERS

  idx_t = input.T.astype(jnp.int32)
  scale_flat = scale.reshape(vocab)
  w_packed = weight_int8.reshape(vocab, 1, dim)

  mesh = plsc.VectorSubcoreMesh(core_axis_name="c", subcore_axis_name="s")
  run = pl.kernel(
      _dequant_kernel,
      out_type=jax.ShapeDtypeStruct(
          (hist, dim // 8, _NUM_WORKERS, 8, bm), jnp.float32),
      mesh=mesh,
      compiler_params=pltpu.CompilerParams(
          needs_layout_passes=False, use_tc_tiling_on_sc=False),
      scratch_types=[
          pltpu.VMEM((hist, bm), jnp.int32),
          pltpu.VMEM((bm, 1, dim), jnp.int8),
          pltpu.VMEM((bm, 1, dim), jnp.int8),
          pltpu.VMEM((bm + _LANES,), jnp.float32),
          pltpu.VMEM((bm + _LANES,), jnp.float32),
          pltpu.VMEM((dim // 8, 8, bm), jnp.float32),
          pltpu.VMEM((dim // 8, 8, bm), jnp.float32),
          pltpu.SemaphoreType.DMA,
          pltpu.SemaphoreType.DMA,
          pltpu.SemaphoreType.DMA,
          pltpu.SemaphoreType.DMA,
      ],
  )
  out5 = run(idx_t, w_packed, scale_flat)
  return out5.transpose(2, 4, 0, 1, 3).reshape(batch, hist, dim)
